# baseline (speedup 1.0000x reference)
import functools

import jax
import jax.numpy as jnp
from jax import lax
from jax.experimental import pallas as pl
from jax.experimental.pallas import tpu as pltpu
from jax.experimental.pallas import tpu_sc as plsc

_N = 1_000_000
_NDAYS = 250
_R = 4
_SCALING = 12.0
_NTILES = 32
_KOFF = _N // _R
_ACCW = 257

_JW = 51200
_NBLK = -(-_N // _JW)


def _dense_body(x_ref, t_ref, y0, y1, y2, y3):
    x4 = x_ref[pl.ds(0, _R), :]
    t4 = t_ref[pl.ds(0, _R), :]
    ht = 0.5 * t4
    y = ht + ht * jnp.tanh((0.5 * _SCALING) * x4)
    for c, yc in enumerate((y0, y1, y2, y3)):
        yc[...] = y[c]


_dense = pl.pallas_call(
    _dense_body,
    grid=(_NBLK,),
    in_specs=[
        pl.BlockSpec((5, _JW), lambda b: (0, b)),
        pl.BlockSpec((5, _JW), lambda b: (0, b)),
    ],
    out_specs=[pl.BlockSpec((_JW,), lambda b: (b,)) for _ in range(4)],
    out_shape=[jax.ShapeDtypeStruct((_N,), jnp.float32) for _ in range(4)],
)

_CH = 8000
_ROWS = _CH // _R
_NCHUNK = _N // _CH
_CPT = -(-_NCHUNK // _NTILES)
_SPAD = 2004

_mesh = plsc.VectorSubcoreMesh(
    core_axis_name="c", subcore_axis_name="s", num_cores=2, num_subcores=16)


@functools.partial(
    pl.kernel,
    out_type=jax.ShapeDtypeStruct((_NTILES, 256), jnp.float32),
    mesh=_mesh,
    compiler_params=pltpu.CompilerParams(
        needs_layout_passes=False, use_tc_tiling_on_sc=False),
    scratch_types=[
        pltpu.VMEM((16, _SPAD), jnp.float32),
        pltpu.VMEM((16, _SPAD), jnp.float32),
        pltpu.VMEM((_CH,), jnp.float32),
        pltpu.VMEM((_CH,), jnp.float32),
        pltpu.VMEM((_CH,), jnp.int32),
        pltpu.VMEM((_CH,), jnp.int32),
        pltpu.VMEM((16 * _ACCW,), jnp.float32),
        pltpu.VMEM((256,), jnp.float32),
        pltpu.SemaphoreType.DMA,
        pltpu.SemaphoreType.DMA,
    ],
)
def _sc_scatter(y0_hbm, y1_hbm, y2_hbm, y3_hbm, w_hbm, d_hbm, out_hbm,
                yb0, yb1, wb0, wb1, db0, db1, acc, redb, sem0, sem1):
    wid = lax.axis_index("s") * 2 + lax.axis_index("c")

    lane = lax.iota(jnp.int32, 16)
    crow = lane % _R
    ccol = lane // _R
    lane_row = lane * _ACCW

    ysl = (y0_hbm, y1_hbm, y2_hbm, y3_hbm)
    sets = ((yb0, wb0, db0, sem0), (yb1, wb1, db1, sem1))

    def _copies(t, bufs):
        u = wid + t * _NTILES
        yb, wb, db, sem = bufs
        cps = []
        for c in range(_R):
            for k in range(_R):
                cps.append(pltpu.make_async_copy(
                    ysl[c].at[pl.ds(k * _KOFF + u * _ROWS, _ROWS)],
                    yb.at[k * _R + c, pl.ds(0, _ROWS)], sem))
        cps.append(pltpu.make_async_copy(w_hbm.at[pl.ds(u * _CH, _CH)], wb, sem))
        cps.append(pltpu.make_async_copy(d_hbm.at[pl.ds(u * _CH, _CH)], db, sem))
        return cps

    def _issue(t, bufs):
        for cp in _copies(t, bufs):
            cp.start()

    def _wait(t, bufs):
        for cp in _copies(t, bufs):
            cp.wait()

    def _compute(bufs):
        yb, wb, db, _ = bufs
        unroll = 10

        def vreg_body(g):
            col = ccol + g * _R
            zv = plsc.load_gather(yb, [crow, col])
            for k in range(1, _R):
                zv = zv + plsc.load_gather(yb, [crow + k * _R, col])
            wv = wb[pl.ds(g * 16, 16)]
            dv = db[pl.ds(g * 16, 16)]
            plsc.addupdate_scatter(acc, [lane_row + dv], wv * zv)

        plsc.parallel_loop(0, _CH // 16, 1, unroll=unroll)(vreg_body)

    zeros16 = jnp.zeros((16,), jnp.float32)

    @plsc.parallel_loop(0, (16 * _ACCW) // 16, 1, unroll=8)
    def zero_body(i):
        acc[pl.ds(i * 16, 16)] = zeros16

    _issue(0, sets[0])

    def outer(o, carry):
        for b in range(2):
            t = o * 2 + b
            u = wid + t * _NTILES
            nxt = t + 1

            @pl.when((wid + nxt * _NTILES < _NCHUNK) & (nxt < _CPT))
            def _():
                _issue(nxt, sets[1 - b])

            @pl.when(u < _NCHUNK)
            def _():
                _wait(t, sets[b])
                _compute(sets[b])

        return carry

    lax.fori_loop(0, _CPT // 2, outer, 0)

    def red_body(jv, carry):
        sv = jnp.zeros((16,), jnp.float32)
        for l in range(16):
            sv = sv + plsc.load_gather(acc, [lane + (l * _ACCW + jv * 16)])
        redb[pl.ds(jv * 16, 16)] = sv
        return carry

    lax.fori_loop(0, 16, red_body, 0)

    pltpu.sync_copy(redb, out_hbm.at[wid])


def _finish_body(p_ref, o_ref):
    pi = jnp.zeros((256,), jnp.float32)
    for r in range(_NTILES):
        pi = pi + p_ref[pl.ds(r * 256, 256)]
    s1 = jnp.sum(pi)
    s2 = jnp.sum(pi * pi)
    loss = -(s1 * jnp.maximum(s1, 0.0)) / s2 / _NDAYS
    o_ref[...] = jnp.full((1, 1), loss, jnp.float32)


_finish = pl.pallas_call(
    _finish_body,
    out_shape=jax.ShapeDtypeStruct((1, 1), jnp.float32),
)


def kernel(inputs, targets, weights, date):
    y0, y1, y2, y3 = _dense(inputs.T, targets.T)
    parts = _sc_scatter(y0, y1, y2, y3, weights, date)
    loss2d = _finish(parts.reshape(-1))
    return loss2d[0, 0]


# speedup vs baseline: 1.0592x; 1.0592x over previous
import functools

import jax
import jax.numpy as jnp
from jax import lax
from jax.experimental import pallas as pl
from jax.experimental.pallas import tpu as pltpu
from jax.experimental.pallas import tpu_sc as plsc

_N = 1_000_000
_NDAYS = 250
_R = 4
_SCALING = 12.0
_NTILES = 32
_KOFF = _N // _R
_ACCW = 257

_JW = 122880
_NBLK = -(-_N // _JW)


def _dense_body(x_ref, t_ref, y0, y1, y2, y3):
    x4 = x_ref[pl.ds(0, _R), :]
    t4 = t_ref[pl.ds(0, _R), :]
    ht = 0.5 * t4
    y = ht + ht * jnp.tanh((0.5 * _SCALING) * x4)
    for c, yc in enumerate((y0, y1, y2, y3)):
        yc[...] = y[c]


_dense = pl.pallas_call(
    _dense_body,
    grid=(_NBLK,),
    in_specs=[
        pl.BlockSpec((5, _JW), lambda b: (0, b)),
        pl.BlockSpec((5, _JW), lambda b: (0, b)),
    ],
    out_specs=[pl.BlockSpec((_JW,), lambda b: (b,)) for _ in range(4)],
    out_shape=[jax.ShapeDtypeStruct((_N,), jnp.float32) for _ in range(4)],
)

_CH = 8000
_ROWS = _CH // _R
_NCHUNK = _N // _CH
_CPT = -(-_NCHUNK // _NTILES)
_SPAD = 2004

_mesh = plsc.VectorSubcoreMesh(
    core_axis_name="c", subcore_axis_name="s", num_cores=2, num_subcores=16)


@functools.partial(
    pl.kernel,
    out_type=jax.ShapeDtypeStruct((_NTILES, 256), jnp.float32),
    mesh=_mesh,
    compiler_params=pltpu.CompilerParams(
        needs_layout_passes=False, use_tc_tiling_on_sc=False),
    scratch_types=[
        pltpu.VMEM((16, _SPAD), jnp.float32),
        pltpu.VMEM((16, _SPAD), jnp.float32),
        pltpu.VMEM((_CH,), jnp.float32),
        pltpu.VMEM((_CH,), jnp.float32),
        pltpu.VMEM((_CH,), jnp.int32),
        pltpu.VMEM((_CH,), jnp.int32),
        pltpu.VMEM((16 * _ACCW,), jnp.float32),
        pltpu.VMEM((256,), jnp.float32),
        pltpu.SemaphoreType.DMA,
        pltpu.SemaphoreType.DMA,
    ],
)
def _sc_scatter(y0_hbm, y1_hbm, y2_hbm, y3_hbm, w_hbm, d_hbm, out_hbm,
                yb0, yb1, wb0, wb1, db0, db1, acc, redb, sem0, sem1):
    wid = lax.axis_index("s") * 2 + lax.axis_index("c")

    lane = lax.iota(jnp.int32, 16)
    crow = lane % _R
    ccol = lane // _R
    lane_row = lane * _ACCW

    ysl = (y0_hbm, y1_hbm, y2_hbm, y3_hbm)
    sets = ((yb0, wb0, db0, sem0), (yb1, wb1, db1, sem1))

    def _copies(t, bufs):
        u = wid + t * _NTILES
        yb, wb, db, sem = bufs
        cps = []
        for c in range(_R):
            for k in range(_R):
                cps.append(pltpu.make_async_copy(
                    ysl[c].at[pl.ds(k * _KOFF + u * _ROWS, _ROWS)],
                    yb.at[k * _R + c, pl.ds(0, _ROWS)], sem))
        cps.append(pltpu.make_async_copy(w_hbm.at[pl.ds(u * _CH, _CH)], wb, sem))
        cps.append(pltpu.make_async_copy(d_hbm.at[pl.ds(u * _CH, _CH)], db, sem))
        return cps

    def _issue(t, bufs):
        for cp in _copies(t, bufs):
            cp.start()

    def _wait(t, bufs):
        for cp in _copies(t, bufs):
            cp.wait()

    def _compute(bufs):
        yb, wb, db, _ = bufs
        unroll = 10

        def vreg_body(g):
            col = ccol + g * _R
            zv = plsc.load_gather(yb, [crow, col])
            for k in range(1, _R):
                zv = zv + plsc.load_gather(yb, [crow + k * _R, col])
            wv = wb[pl.ds(g * 16, 16)]
            dv = db[pl.ds(g * 16, 16)]
            plsc.addupdate_scatter(acc, [lane_row + dv], wv * zv)

        plsc.parallel_loop(0, _CH // 16, 1, unroll=unroll)(vreg_body)

    zeros16 = jnp.zeros((16,), jnp.float32)

    @plsc.parallel_loop(0, (16 * _ACCW) // 16, 1, unroll=8)
    def zero_body(i):
        acc[pl.ds(i * 16, 16)] = zeros16

    _issue(0, sets[0])

    def outer(o, carry):
        for b in range(2):
            t = o * 2 + b
            u = wid + t * _NTILES
            nxt = t + 1

            @pl.when((wid + nxt * _NTILES < _NCHUNK) & (nxt < _CPT))
            def _():
                _issue(nxt, sets[1 - b])

            @pl.when(u < _NCHUNK)
            def _():
                _wait(t, sets[b])
                _compute(sets[b])

        return carry

    lax.fori_loop(0, _CPT // 2, outer, 0)

    def red_body(jv, carry):
        sv = jnp.zeros((16,), jnp.float32)
        for l in range(16):
            sv = sv + plsc.load_gather(acc, [lane + (l * _ACCW + jv * 16)])
        redb[pl.ds(jv * 16, 16)] = sv
        return carry

    lax.fori_loop(0, 16, red_body, 0)

    pltpu.sync_copy(redb, out_hbm.at[wid])


def _finish_body(p_ref, o_ref):
    pi = jnp.zeros((256,), jnp.float32)
    for r in range(_NTILES):
        pi = pi + p_ref[pl.ds(r * 256, 256)]
    s1 = jnp.sum(pi)
    s2 = jnp.sum(pi * pi)
    loss = -(s1 * jnp.maximum(s1, 0.0)) / s2 / _NDAYS
    o_ref[...] = jnp.full((1, 1), loss, jnp.float32)


_finish = pl.pallas_call(
    _finish_body,
    out_shape=jax.ShapeDtypeStruct((1, 1), jnp.float32),
)


def kernel(inputs, targets, weights, date):
    y0, y1, y2, y3 = _dense(inputs.T, targets.T)
    parts = _sc_scatter(y0, y1, y2, y3, weights, date)
    loss2d = _finish(parts.reshape(-1))
    return loss2d[0, 0]
